# baseline (speedup 1.0000x reference)
import functools

import jax
import jax.numpy as jnp
from jax import lax
from jax.experimental import pallas as pl
from jax.experimental.pallas import tpu as pltpu
from jax.experimental.pallas import tpu_sc as plsc

_CUTOFF = 5.0
_NA = 10000
_NBH = 32
_NB = 25
_F = 128
_E = _NA * _NBH

_LOG2 = 0.6931471805599453


def _ssp(t):
    return jnp.maximum(t, 0.0) + jnp.log1p(jnp.exp(-jnp.abs(t))) - _LOG2


_KY_BLK = 2000


def _ky_body(x_ref, w_ref, o_ref):
    o_ref[...] = jnp.dot(x_ref[...], w_ref[...],
                         preferred_element_type=jnp.float32)


def _ky(x2d, w):
    return pl.pallas_call(
        _ky_body,
        grid=(_NA // _KY_BLK,),
        in_specs=[
            pl.BlockSpec((_KY_BLK, _F), lambda i: (i, 0)),
            pl.BlockSpec((_F, _F), lambda i: (0, 0)),
        ],
        out_specs=pl.BlockSpec((_KY_BLK, _F), lambda i: (i, 0)),
        out_shape=jax.ShapeDtypeStruct((_NA, _F), jnp.float32),
    )(x2d, w)


_KF_ABLK = 200
_KF_EBLK = _KF_ABLK * _NBH


def _kf_body(fij_ref, r_ref, m_ref, w1_ref, b1_ref, w2_ref, b2_ref, o_ref):
    h = _ssp(jnp.dot(fij_ref[...], w1_ref[...],
                     preferred_element_type=jnp.float32) + b1_ref[...])
    wf = jnp.dot(h, w2_ref[...], preferred_element_type=jnp.float32) + b2_ref[...]
    r = r_ref[...]
    c = 0.5 * (jnp.cos(r * (jnp.pi / _CUTOFF)) + 1.0)
    c = c * (r < _CUTOFF).astype(jnp.float32) * m_ref[...]
    c_rep = lax.broadcast_in_dim(c, (_KF_ABLK, _NBH, _NBH), (0, 2))
    c_rep = c_rep.reshape(_KF_EBLK, _NBH)
    row_n = lax.broadcasted_iota(jnp.int32, (_KF_EBLK, _NBH), 0) % _NBH
    lane = lax.broadcasted_iota(jnp.int32, (_KF_EBLK, _NBH), 1)
    cc = jnp.sum(jnp.where(row_n == lane, c_rep, 0.0), axis=1, keepdims=True)
    o_ref[...] = wf * cc


def _kf(fij2d, r2, m2, w1, b1, w2, b2):
    return pl.pallas_call(
        _kf_body,
        grid=(_NA // _KF_ABLK,),
        in_specs=[
            pl.BlockSpec((_KF_EBLK, _NB), lambda i: (i, 0)),
            pl.BlockSpec((_KF_ABLK, _NBH), lambda i: (i, 0)),
            pl.BlockSpec((_KF_ABLK, _NBH), lambda i: (i, 0)),
            pl.BlockSpec((_NB, _F), lambda i: (0, 0)),
            pl.BlockSpec((1, _F), lambda i: (0, 0)),
            pl.BlockSpec((_F, _F), lambda i: (0, 0)),
            pl.BlockSpec((1, _F), lambda i: (0, 0)),
        ],
        out_specs=pl.BlockSpec((_KF_EBLK, _F), lambda i: (i, 0)),
        out_shape=jax.ShapeDtypeStruct((_E, _F), jnp.float32),
    )(fij2d, r2, m2, w1, b1, w2, b2)


_NC, _NS = 2, 16
_NW = _NC * _NS
_APW = 320
_NAP = _NW * _APW
_ACH = 4
_ECH = _ACH * _NBH
_NCH = _APW // _ACH
_IPW = _APW * _NBH


def _sc_reduce(y, wf, idx_pad):
    mesh = plsc.VectorSubcoreMesh(core_axis_name="c", subcore_axis_name="s")

    @functools.partial(
        pl.kernel,
        mesh=mesh,
        out_type=jax.ShapeDtypeStruct((_NAP, _F), jnp.float32),
        scratch_types=[
            pltpu.VMEM((_IPW,), jnp.int32),
            pltpu.VMEM((2, _ECH, _F), jnp.float32),
            pltpu.VMEM((2, _ECH, _F), jnp.float32),
            pltpu.VMEM((2, _ACH, _F), jnp.float32),
            pltpu.SemaphoreType.DMA((2,)),
            pltpu.SemaphoreType.DMA((2,)),
            pltpu.SemaphoreType.DMA((2,)),
        ],
    )
    def kr(y_hbm, wf_hbm, idx_hbm, out_hbm, idx_v, rows_v, wfb_v, acc_v,
           gsem, wsem, osem):
        wid = lax.axis_index("s") * _NC + lax.axis_index("c")
        a0 = wid * _APW
        e0 = a0 * _NBH

        pltpu.sync_copy(idx_hbm.at[pl.ds(e0, _IPW)], idx_v)

        def issue(c, buf):
            ew = e0 + c * _ECH
            ew_wf = jnp.minimum(ew, _E - _ECH)
            pltpu.async_copy(
                y_hbm.at[idx_v.at[pl.ds(c * _ECH, _ECH)]],
                rows_v.at[buf], gsem.at[buf])
            pltpu.async_copy(
                wf_hbm.at[pl.ds(ew_wf, _ECH), :],
                wfb_v.at[buf], wsem.at[buf])

        issue(0, 0)

        def body(c, carry):
            buf = lax.rem(c, 2)
            nbuf = 1 - buf

            @pl.when(c < _NCH - 1)
            def _():
                issue(c + 1, nbuf)

            pltpu.make_async_copy(
                y_hbm.at[idx_v.at[pl.ds(c * _ECH, _ECH)]],
                rows_v.at[buf], gsem.at[buf]).wait()
            pltpu.make_async_copy(
                wf_hbm.at[pl.ds(0, _ECH), :],
                wfb_v.at[buf], wsem.at[buf]).wait()

            @pl.when(c >= 2)
            def _():
                pltpu.make_async_copy(
                    acc_v.at[buf],
                    out_hbm.at[pl.ds(a0, _ACH), :], osem.at[buf]).wait()

            for a in range(_ACH):
                def nb4(i, acc):
                    base = a * _NBH + i * 4
                    for dn in range(4):
                        row = base + dn
                        acc = tuple(
                            acc[k]
                            + rows_v[buf, row, pl.ds(k * 16, 16)]
                            * wfb_v[buf, row, pl.ds(k * 16, 16)]
                            for k in range(8))
                    return acc
                acc = lax.fori_loop(
                    0, _NBH // 4, nb4,
                    tuple(jnp.zeros((16,), jnp.float32) for _ in range(8)))
                for k in range(8):
                    acc_v[buf, a, pl.ds(k * 16, 16)] = acc[k]

            pltpu.async_copy(
                acc_v.at[buf],
                out_hbm.at[pl.ds(a0 + c * _ACH, _ACH), :], osem.at[buf])
            return carry

        lax.fori_loop(0, _NCH, body, 0)

        for buf in range(2):
            pltpu.make_async_copy(
                acc_v.at[buf],
                out_hbm.at[pl.ds(a0, _ACH), :], osem.at[buf]).wait()

    return kr(y, wf, idx_pad)


_KO_BLK = 2000


def _ko_body(agg_ref, wfo_ref, bfo_ref, wd_ref, bd_ref, o_ref):
    v = _ssp(jnp.dot(agg_ref[...], wfo_ref[...],
                     preferred_element_type=jnp.float32) + bfo_ref[...])
    o_ref[...] = jnp.dot(v, wd_ref[...],
                         preferred_element_type=jnp.float32) + bd_ref[...]


def _ko(agg, wfo, bfo, wd, bd):
    return pl.pallas_call(
        _ko_body,
        grid=(_NA // _KO_BLK,),
        in_specs=[
            pl.BlockSpec((_KO_BLK, _F), lambda i: (i, 0)),
            pl.BlockSpec((_F, _F), lambda i: (0, 0)),
            pl.BlockSpec((1, _F), lambda i: (0, 0)),
            pl.BlockSpec((_F, _F), lambda i: (0, 0)),
            pl.BlockSpec((1, _F), lambda i: (0, 0)),
        ],
        out_specs=pl.BlockSpec((_KO_BLK, _F), lambda i: (i, 0)),
        out_shape=jax.ShapeDtypeStruct((_NA, _F), jnp.float32),
    )(agg, wfo, bfo, wd, bd)


def kernel(x, r_ij, neighbors, neighbor_mask, f_ij,
           W_f1, b_f1, W_f2, b_f2, W_in2f, W_f2out, b_f2out, W_dense, b_dense):
    B = x.shape[0]
    x2d = x.reshape(_NA, _F)
    fij2d = f_ij.reshape(_E, _NB)
    r2 = r_ij.reshape(_NA, _NBH)
    m2 = neighbor_mask.reshape(_NA, _NBH)
    idx = neighbors.reshape(_E).astype(jnp.int32)
    idx_pad = jnp.pad(idx, (0, _NAP * _NBH - _E))

    y = _ky(x2d, W_in2f)
    wf = _kf(fij2d, r2, m2, W_f1, b_f1.reshape(1, _F), W_f2, b_f2.reshape(1, _F))
    agg = _sc_reduce(y, wf, idx_pad)[: _NA]
    v = _ko(agg, W_f2out, b_f2out.reshape(1, _F), W_dense, b_dense.reshape(1, _F))
    return v.reshape(B, _NA, _F)


# speedup vs baseline: 1.8698x; 1.8698x over previous
import functools

import jax
import jax.numpy as jnp
from jax import lax
from jax.experimental import pallas as pl
from jax.experimental.pallas import tpu as pltpu
from jax.experimental.pallas import tpu_sc as plsc

_CUTOFF = 5.0
_NA = 10000
_NBH = 32
_NB = 25
_F = 128
_E = _NA * _NBH

_LOG2 = 0.6931471805599453


def _ssp(t):
    return jnp.maximum(t, 0.0) + jnp.log1p(jnp.exp(-jnp.abs(t))) - _LOG2


_KY_BLK = 2000


def _ky_body(x_ref, w_ref, o_ref):
    o_ref[...] = jnp.dot(x_ref[...], w_ref[...],
                         preferred_element_type=jnp.float32)


def _ky(x2d, w):
    return pl.pallas_call(
        _ky_body,
        grid=(_NA // _KY_BLK,),
        in_specs=[
            pl.BlockSpec((_KY_BLK, _F), lambda i: (i, 0)),
            pl.BlockSpec((_F, _F), lambda i: (0, 0)),
        ],
        out_specs=pl.BlockSpec((_KY_BLK, _F), lambda i: (i, 0)),
        out_shape=jax.ShapeDtypeStruct((_NA, _F), jnp.float32),
    )(x2d, w)


_KF_ABLK = 200
_KF_EBLK = _KF_ABLK * _NBH


def _kf_body(fij_ref, r_ref, m_ref, w1_ref, b1_ref, w2_ref, b2_ref, o_ref):
    h = _ssp(jnp.dot(fij_ref[...], w1_ref[...],
                     preferred_element_type=jnp.float32) + b1_ref[...])
    wf = jnp.dot(h, w2_ref[...], preferred_element_type=jnp.float32) + b2_ref[...]
    r = r_ref[...]
    c = 0.5 * (jnp.cos(r * (jnp.pi / _CUTOFF)) + 1.0)
    c = c * (r < _CUTOFF).astype(jnp.float32) * m_ref[...]
    c_rep = lax.broadcast_in_dim(c, (_KF_ABLK, _NBH, _NBH), (0, 2))
    c_rep = c_rep.reshape(_KF_EBLK, _NBH)
    row_n = lax.broadcasted_iota(jnp.int32, (_KF_EBLK, _NBH), 0) % _NBH
    lane = lax.broadcasted_iota(jnp.int32, (_KF_EBLK, _NBH), 1)
    cc = jnp.sum(jnp.where(row_n == lane, c_rep, 0.0), axis=1, keepdims=True)
    o_ref[...] = wf * cc


def _kf(fij2d, r2, m2, w1, b1, w2, b2):
    return pl.pallas_call(
        _kf_body,
        grid=(_NA // _KF_ABLK,),
        in_specs=[
            pl.BlockSpec((_KF_EBLK, _NB), lambda i: (i, 0)),
            pl.BlockSpec((_KF_ABLK, _NBH), lambda i: (i, 0)),
            pl.BlockSpec((_KF_ABLK, _NBH), lambda i: (i, 0)),
            pl.BlockSpec((_NB, _F), lambda i: (0, 0)),
            pl.BlockSpec((1, _F), lambda i: (0, 0)),
            pl.BlockSpec((_F, _F), lambda i: (0, 0)),
            pl.BlockSpec((1, _F), lambda i: (0, 0)),
        ],
        out_specs=pl.BlockSpec((_KF_EBLK, _F), lambda i: (i, 0)),
        out_shape=jax.ShapeDtypeStruct((_E, _F), jnp.float32),
    )(fij2d, r2, m2, w1, b1, w2, b2)


_NC, _NS = 2, 16
_NW = _NC * _NS
_APW = 320
_NAP = _NW * _APW
_ACH = 4
_ECH = _ACH * _NBH
_NCH = _APW // _ACH
_IPW = _APW * _NBH


def _sc_reduce(y, wf, idx_pad):
    mesh = plsc.VectorSubcoreMesh(core_axis_name="c", subcore_axis_name="s")

    @functools.partial(
        pl.kernel,
        mesh=mesh,
        out_type=jax.ShapeDtypeStruct((_NAP, _F), jnp.float32),
        scratch_types=[
            pltpu.VMEM((_IPW,), jnp.int32),
            pltpu.VMEM((2, _ECH, _F), jnp.float32),
            pltpu.VMEM((2, _ECH, _F), jnp.float32),
            pltpu.VMEM((2, _ACH, _F), jnp.float32),
            pltpu.SemaphoreType.DMA((2,)),
            pltpu.SemaphoreType.DMA((2,)),
            pltpu.SemaphoreType.DMA((2,)),
        ],
    )
    def kr(y_hbm, wf_hbm, idx_hbm, out_hbm, idx_v, rows_v, wfb_v, acc_v,
           gsem, wsem, osem):
        wid = lax.axis_index("s") * _NC + lax.axis_index("c")
        a0 = wid * _APW
        e0 = a0 * _NBH

        pltpu.sync_copy(idx_hbm.at[pl.ds(e0, _IPW)], idx_v)

        def issue(c, buf):
            ew = e0 + c * _ECH
            ew_wf = jnp.minimum(ew, _E - _ECH)
            pltpu.async_copy(
                y_hbm.at[idx_v.at[pl.ds(c * _ECH, _ECH)]],
                rows_v.at[buf], gsem.at[buf])
            pltpu.async_copy(
                wf_hbm.at[pl.ds(ew_wf, _ECH), :],
                wfb_v.at[buf], wsem.at[buf])

        issue(0, 0)

        def body(c, carry):
            buf = lax.rem(c, 2)
            nbuf = 1 - buf

            @pl.when(c < _NCH - 1)
            def _():
                issue(c + 1, nbuf)

            pltpu.make_async_copy(
                y_hbm.at[idx_v.at[pl.ds(c * _ECH, _ECH)]],
                rows_v.at[buf], gsem.at[buf]).wait()
            pltpu.make_async_copy(
                wf_hbm.at[pl.ds(0, _ECH), :],
                wfb_v.at[buf], wsem.at[buf]).wait()

            @pl.when(c >= 2)
            def _():
                pltpu.make_async_copy(
                    acc_v.at[buf],
                    out_hbm.at[pl.ds(a0, _ACH), :], osem.at[buf]).wait()

            for a in range(_ACH):
                def nb4(i, acc):
                    base = a * _NBH + i * 4
                    for dn in range(4):
                        row = base + dn
                        acc = tuple(
                            acc[k]
                            + rows_v[buf, row, pl.ds(k * 16, 16)]
                            * wfb_v[buf, row, pl.ds(k * 16, 16)]
                            for k in range(8))
                    return acc
                acc = lax.fori_loop(
                    0, _NBH // 4, nb4,
                    tuple(jnp.zeros((16,), jnp.float32) for _ in range(8)))
                for k in range(8):
                    acc_v[buf, a, pl.ds(k * 16, 16)] = acc[k]

            pltpu.async_copy(
                acc_v.at[buf],
                out_hbm.at[pl.ds(a0 + c * _ACH, _ACH), :], osem.at[buf])
            return carry

        lax.fori_loop(0, _NCH, body, 0)

        for buf in range(2):
            pltpu.make_async_copy(
                acc_v.at[buf],
                out_hbm.at[pl.ds(a0, _ACH), :], osem.at[buf]).wait()

    return kr(y, wf, idx_pad)


_KO_BLK = 2000


def _ko_body(agg_ref, wfo_ref, bfo_ref, wd_ref, bd_ref, o_ref):
    v = _ssp(jnp.dot(agg_ref[...], wfo_ref[...],
                     preferred_element_type=jnp.float32) + bfo_ref[...])
    o_ref[...] = jnp.dot(v, wd_ref[...],
                         preferred_element_type=jnp.float32) + bd_ref[...]


def _ko(agg, wfo, bfo, wd, bd):
    return pl.pallas_call(
        _ko_body,
        grid=(_NA // _KO_BLK,),
        in_specs=[
            pl.BlockSpec((_KO_BLK, _F), lambda i: (i, 0)),
            pl.BlockSpec((_F, _F), lambda i: (0, 0)),
            pl.BlockSpec((1, _F), lambda i: (0, 0)),
            pl.BlockSpec((_F, _F), lambda i: (0, 0)),
            pl.BlockSpec((1, _F), lambda i: (0, 0)),
        ],
        out_specs=pl.BlockSpec((_KO_BLK, _F), lambda i: (i, 0)),
        out_shape=jax.ShapeDtypeStruct((_NA, _F), jnp.float32),
    )(agg, wfo, bfo, wd, bd)


def kernel(x, r_ij, neighbors, neighbor_mask, f_ij,
           W_f1, b_f1, W_f2, b_f2, W_in2f, W_f2out, b_f2out, W_dense, b_dense):
    B = x.shape[0]
    x2d = x.reshape(_NA, _F)
    fij2d = f_ij.reshape(_E, _NB)
    r2 = r_ij.reshape(_NA, _NBH)
    m2 = neighbor_mask.reshape(_NA, _NBH)
    idx = neighbors.reshape(_E).astype(jnp.int32)
    n_pad = _NAP * _NBH - _E
    idx_pad = jnp.concatenate(
        [idx, (jnp.arange(n_pad, dtype=jnp.int32) % _NA)])

    y = _ky(x2d, W_in2f)
    wf = _kf(fij2d, r2, m2, W_f1, b_f1.reshape(1, _F), W_f2, b_f2.reshape(1, _F))
    agg = _sc_reduce(y, wf, idx_pad)[: _NA]
    v = _ko(agg, W_f2out, b_f2out.reshape(1, _F), W_dense, b_dense.reshape(1, _F))
    return v.reshape(B, _NA, _F)


# speedup vs baseline: 2.0864x; 1.1159x over previous
import functools

import jax
import jax.numpy as jnp
from jax import lax
from jax.experimental import pallas as pl
from jax.experimental.pallas import tpu as pltpu
from jax.experimental.pallas import tpu_sc as plsc

_CUTOFF = 5.0
_NA = 10000
_NBH = 32
_NB = 25
_F = 128
_E = _NA * _NBH

_LOG2 = 0.6931471805599453


def _ssp(t):
    return jnp.maximum(t, 0.0) + jnp.log1p(jnp.exp(-jnp.abs(t))) - _LOG2


_KY_BLK = 2000


def _ky_body(x_ref, w_ref, o_ref):
    o_ref[...] = jnp.dot(x_ref[...], w_ref[...],
                         preferred_element_type=jnp.float32)


def _ky(x2d, w):
    return pl.pallas_call(
        _ky_body,
        grid=(_NA // _KY_BLK,),
        in_specs=[
            pl.BlockSpec((_KY_BLK, _F), lambda i: (i, 0)),
            pl.BlockSpec((_F, _F), lambda i: (0, 0)),
        ],
        out_specs=pl.BlockSpec((_KY_BLK, _F), lambda i: (i, 0)),
        out_shape=jax.ShapeDtypeStruct((_NA, _F), jnp.float32),
    )(x2d, w)


_KF_ABLK = 80
_KF_EBLK = _KF_ABLK * _NBH


def _kf_body(fij_ref, r_ref, m_ref, w1_ref, b1_ref, w2_ref, b2_ref, o_ref):
    h = _ssp(jnp.dot(fij_ref[...], w1_ref[...],
                     preferred_element_type=jnp.float32) + b1_ref[...])
    wf = jnp.dot(h, w2_ref[...], preferred_element_type=jnp.float32) + b2_ref[...]
    r = r_ref[...]
    c = 0.5 * (jnp.cos(r * (jnp.pi / _CUTOFF)) + 1.0)
    c = c * (r < _CUTOFF).astype(jnp.float32) * m_ref[...]
    c_rep = lax.broadcast_in_dim(c, (_KF_ABLK, _NBH, _NBH), (0, 2))
    c_rep = c_rep.reshape(_KF_EBLK, _NBH)
    row_n = lax.broadcasted_iota(jnp.int32, (_KF_EBLK, _NBH), 0) % _NBH
    lane = lax.broadcasted_iota(jnp.int32, (_KF_EBLK, _NBH), 1)
    cc = jnp.sum(jnp.where(row_n == lane, c_rep, 0.0), axis=1, keepdims=True)
    o_ref[...] = wf * cc


def _kf(fij2d, r2, m2, w1, b1, w2, b2):
    na = r2.shape[0]
    e = na * _NBH
    return pl.pallas_call(
        _kf_body,
        grid=(na // _KF_ABLK,),
        in_specs=[
            pl.BlockSpec((_KF_EBLK, _NB), lambda i: (i, 0)),
            pl.BlockSpec((_KF_ABLK, _NBH), lambda i: (i, 0)),
            pl.BlockSpec((_KF_ABLK, _NBH), lambda i: (i, 0)),
            pl.BlockSpec((_NB, _F), lambda i: (0, 0)),
            pl.BlockSpec((1, _F), lambda i: (0, 0)),
            pl.BlockSpec((_F, _F), lambda i: (0, 0)),
            pl.BlockSpec((1, _F), lambda i: (0, 0)),
        ],
        out_specs=pl.BlockSpec((_KF_EBLK, _F), lambda i: (i, 0)),
        out_shape=jax.ShapeDtypeStruct((e, _F), jnp.float32),
    )(fij2d, r2, m2, w1, b1, w2, b2)


_NC, _NS = 2, 16
_NW = _NC * _NS
_ACH = 4
_ECH = _ACH * _NBH


def _sc_reduce(y, wf, idx_half, apw):
    nap = _NW * apw
    ipw = apw * _NBH
    nch = apw // _ACH
    e_wf = wf.shape[0]
    mesh = plsc.VectorSubcoreMesh(core_axis_name="c", subcore_axis_name="s")

    @functools.partial(
        pl.kernel,
        mesh=mesh,
        out_type=jax.ShapeDtypeStruct((nap, _F), jnp.float32),
        scratch_types=[
            pltpu.VMEM((ipw,), jnp.int32),
            pltpu.VMEM((2, _ECH, _F), jnp.float32),
            pltpu.VMEM((2, _ECH, _F), jnp.float32),
            pltpu.VMEM((2, _ACH, _F), jnp.float32),
            pltpu.SemaphoreType.DMA((2,)),
            pltpu.SemaphoreType.DMA((2,)),
            pltpu.SemaphoreType.DMA((2,)),
        ],
    )
    def kr(y_hbm, wf_hbm, idx_hbm, out_hbm, idx_v, rows_v, wfb_v, acc_v,
           gsem, wsem, osem):
        wid = lax.axis_index("s") * _NC + lax.axis_index("c")
        a0 = wid * apw
        e0 = a0 * _NBH

        pltpu.sync_copy(idx_hbm.at[pl.ds(e0, ipw)], idx_v)

        def issue(c, buf):
            ew = e0 + c * _ECH
            ew_wf = jnp.minimum(ew, e_wf - _ECH)
            pltpu.async_copy(
                y_hbm.at[idx_v.at[pl.ds(c * _ECH, _ECH)]],
                rows_v.at[buf], gsem.at[buf])
            pltpu.async_copy(
                wf_hbm.at[pl.ds(ew_wf, _ECH), :],
                wfb_v.at[buf], wsem.at[buf])

        issue(0, 0)

        def body(c, carry):
            buf = lax.rem(c, 2)
            nbuf = 1 - buf

            @pl.when(c < nch - 1)
            def _():
                issue(c + 1, nbuf)

            pltpu.make_async_copy(
                y_hbm.at[idx_v.at[pl.ds(c * _ECH, _ECH)]],
                rows_v.at[buf], gsem.at[buf]).wait()
            pltpu.make_async_copy(
                wf_hbm.at[pl.ds(0, _ECH), :],
                wfb_v.at[buf], wsem.at[buf]).wait()

            @pl.when(c >= 2)
            def _():
                pltpu.make_async_copy(
                    acc_v.at[buf],
                    out_hbm.at[pl.ds(a0, _ACH), :], osem.at[buf]).wait()

            for a in range(_ACH):
                def nb4(i, acc):
                    base = a * _NBH + i * 4
                    for dn in range(4):
                        row = base + dn
                        acc = tuple(
                            acc[k]
                            + rows_v[buf, row, pl.ds(k * 16, 16)]
                            * wfb_v[buf, row, pl.ds(k * 16, 16)]
                            for k in range(8))
                    return acc
                acc = lax.fori_loop(
                    0, _NBH // 4, nb4,
                    tuple(jnp.zeros((16,), jnp.float32) for _ in range(8)))
                for k in range(8):
                    acc_v[buf, a, pl.ds(k * 16, 16)] = acc[k]

            pltpu.async_copy(
                acc_v.at[buf],
                out_hbm.at[pl.ds(a0 + c * _ACH, _ACH), :], osem.at[buf])
            return carry

        lax.fori_loop(0, nch, body, 0)

        for buf in range(2):
            pltpu.make_async_copy(
                acc_v.at[buf],
                out_hbm.at[pl.ds(a0, _ACH), :], osem.at[buf]).wait()

    return kr(y, wf, idx_half)


_KO_BLK = 1280


def _ko_body(agg_ref, wfo_ref, bfo_ref, wd_ref, bd_ref, o_ref):
    v = _ssp(jnp.dot(agg_ref[...], wfo_ref[...],
                     preferred_element_type=jnp.float32) + bfo_ref[...])
    o_ref[...] = jnp.dot(v, wd_ref[...],
                         preferred_element_type=jnp.float32) + bd_ref[...]


def _ko(agg, wfo, bfo, wd, bd):
    na = agg.shape[0]
    return pl.pallas_call(
        _ko_body,
        grid=(na // _KO_BLK,),
        in_specs=[
            pl.BlockSpec((_KO_BLK, _F), lambda i: (i, 0)),
            pl.BlockSpec((_F, _F), lambda i: (0, 0)),
            pl.BlockSpec((1, _F), lambda i: (0, 0)),
            pl.BlockSpec((_F, _F), lambda i: (0, 0)),
            pl.BlockSpec((1, _F), lambda i: (0, 0)),
        ],
        out_specs=pl.BlockSpec((_KO_BLK, _F), lambda i: (i, 0)),
        out_shape=jax.ShapeDtypeStruct((na, _F), jnp.float32),
    )(agg, wfo, bfo, wd, bd)


_HALF = 5120
_APW_H = _HALF // _NW


def kernel(x, r_ij, neighbors, neighbor_mask, f_ij,
           W_f1, b_f1, W_f2, b_f2, W_in2f, W_f2out, b_f2out, W_dense, b_dense):
    B = x.shape[0]
    x2d = x.reshape(_NA, _F)
    b1 = b_f1.reshape(1, _F)
    b2 = b_f2.reshape(1, _F)
    bfo = b_f2out.reshape(1, _F)
    bd = b_dense.reshape(1, _F)

    idx = neighbors.reshape(_E).astype(jnp.int32)
    n_pad = 2 * _HALF * _NBH - _E
    idx_pad = jnp.concatenate(
        [idx, (jnp.arange(n_pad, dtype=jnp.int32) % _NA)])

    y = _ky(x2d, W_in2f)

    outs = []
    for h in range(2):
        lo = h * _HALF
        hi = min(_NA, lo + _HALF)
        na_h = hi - lo
        fij_h = f_ij[:, lo:hi].reshape(na_h * _NBH, _NB)
        r_h = r_ij[:, lo:hi].reshape(na_h, _NBH)
        m_h = neighbor_mask[:, lo:hi].reshape(na_h, _NBH)
        idx_h = lax.dynamic_slice(idx_pad, (lo * _NBH,), (_HALF * _NBH,))

        wf_h = _kf(fij_h, r_h, m_h, W_f1, b1, W_f2, b2)
        agg_h = _sc_reduce(y, wf_h, idx_h, _APW_H)
        v_h = _ko(agg_h, W_f2out, bfo, W_dense, bd)
        outs.append(v_h[:na_h])

    v = jnp.concatenate(outs, axis=0)
    return v.reshape(B, _NA, _F)
